# baseline (speedup 1.0000x reference)
import functools

import jax
import jax.numpy as jnp
from jax import lax
from jax.experimental import pallas as pl
from jax.experimental.pallas import tpu as pltpu
from jax.experimental.pallas import tpu_sc as plsc

B = 1024
C = 100000
P = 50
EPS = 1e-07

NC = 2
NS = 16
NW = NC * NS
ROWS_PER_W = B // NW
PPAD = 64


def _sc_gather_body(ypred_hbm, yt_hbm, out_hbm, idx_v, row_v, val_v, sem):
    wid = lax.axis_index("s") * NC + lax.axis_index("c")
    row_base = wid * ROWS_PER_W
    pltpu.sync_copy(yt_hbm.at[wid], idx_v)

    for r in range(ROWS_PER_W):
        pltpu.async_copy(ypred_hbm.at[row_base + r], row_v, sem).wait()
        for j in range(PPAD // 16):
            cols = idx_v[r, pl.ds(j * 16, 16)]
            val_v[r, pl.ds(j * 16, 16)] = plsc.load_gather(row_v, [cols])
    pltpu.sync_copy(val_v, out_hbm.at[wid])


_sc_gather = functools.partial(
    pl.kernel,
    out_type=jax.ShapeDtypeStruct((NW, ROWS_PER_W, PPAD), jnp.float32),
    mesh=plsc.VectorSubcoreMesh(core_axis_name="c", subcore_axis_name="s"),
    compiler_params=pltpu.CompilerParams(needs_layout_passes=False),
    scratch_types=[
        pltpu.VMEM((ROWS_PER_W, PPAD), jnp.int32),
        pltpu.VMEM((C,), jnp.float32),
        pltpu.VMEM((ROWS_PER_W, PPAD), jnp.float32),
        pltpu.SemaphoreType.DMA,
    ],
)(_sc_gather_body)


R = 32


def _lse_body(ypred_ref, out_ref):
    x = ypred_ref[...]
    m = jnp.max(x, axis=1, keepdims=True)
    m0 = jnp.maximum(m, 0.0)
    s = jnp.sum(jnp.exp(x - m0), axis=1, keepdims=True)
    out_ref[...] = m0 + jnp.log(s + jnp.exp(-m0))


_lse = pl.pallas_call(
    _lse_body,
    grid=(B // R,),
    in_specs=[pl.BlockSpec((R, C), lambda i: (i, 0))],
    out_specs=pl.BlockSpec((R, 1), lambda i: (i, 0)),
    out_shape=jax.ShapeDtypeStruct((B, 1), jnp.float32),
)


def _combine_body(all_ref, ypos_ref, out_ref):
    all_loss = all_ref[...]
    yp = ypos_ref[:, :P]
    mn = jnp.maximum(jnp.max(-yp, axis=1, keepdims=True), 0.0)
    pos_loss = mn + jnp.log(
        jnp.sum(jnp.exp(-yp - mn), axis=1, keepdims=True) + jnp.exp(-mn))
    mq = jnp.max(yp, axis=1, keepdims=True)
    lse_pos = mq + jnp.log(jnp.sum(jnp.exp(yp - mq), axis=1, keepdims=True))
    aux = jnp.clip(1.0 - jnp.exp(lse_pos - all_loss), EPS, 1.0)
    out_ref[...] = pos_loss + all_loss + jnp.log(aux)


_combine = pl.pallas_call(
    _combine_body,
    in_specs=[
        pl.BlockSpec((B, 1), lambda: (0, 0)),
        pl.BlockSpec((B, PPAD), lambda: (0, 0)),
    ],
    out_specs=pl.BlockSpec((B, 1), lambda: (0, 0)),
    out_shape=jax.ShapeDtypeStruct((B, 1), jnp.float32),
)


def kernel(y_pred, y_true):
    yt = jnp.pad(y_true.astype(jnp.int32), ((0, 0), (0, PPAD - P)))
    ypos = _sc_gather(y_pred, yt.reshape(NW, ROWS_PER_W, PPAD))
    all_loss = _lse(y_pred)
    out = _combine(all_loss, ypos.reshape(B, PPAD))
    return out.reshape(B)
